# Initial kernel scaffold; baseline (speedup 1.0000x reference)
import jax
import jax.numpy as jnp
from jax.experimental import pallas as pl


def kernel(x, edge_index, rel_type, norm, weight, gate_weight):
    raise NotImplementedError("write your pallas kernel here")


# speedup vs baseline: 17.9227x; 17.9227x over previous
import functools

import jax
import jax.numpy as jnp
from jax import lax
from jax.experimental import pallas as pl
from jax.experimental.pallas import tpu as pltpu
from jax.experimental.pallas import tpu_sc as plsc

N = 10000
E = 320000
F = 128
O = 256
R = 8
HALF = O // 2

NC = 2
NS = 16
EPT = E // NS
CH = 80
NCHUNK = EPT // CH
NPAD = 10240
ROWS_PT = NPAD // NS
RB = 128

BN = 1000
NB = N // BN


def _proj_body(x_ref, w_ref, gw_ref, xw_ref):
    xb = x_ref[...]
    w = w_ref[0]
    y = jnp.dot(xb, w, preferred_element_type=jnp.float32)
    gw = gw_ref[0]
    gz = jnp.dot(xb, gw, preferred_element_type=jnp.float32)
    y = y * jax.nn.sigmoid(gz)
    xw_ref[0, 0] = y[:, :HALF]
    xw_ref[1, 0] = y[:, HALF:]


def _projection(x, weight, gate_weight):
    return pl.pallas_call(
        _proj_body,
        grid=(NB, R),
        in_specs=[
            pl.BlockSpec((BN, F), lambda nb, r: (nb, 0)),
            pl.BlockSpec((1, F, O), lambda nb, r: (r, 0, 0)),
            pl.BlockSpec((1, F, 1), lambda nb, r: (r, 0, 0)),
        ],
        out_specs=pl.BlockSpec((2, 1, BN, HALF), lambda nb, r: (0, r, nb, 0)),
        out_shape=jax.ShapeDtypeStruct((2, R, N, HALF), jnp.float32),
    )(x, weight, gate_weight)


_sc_mesh = plsc.VectorSubcoreMesh(
    core_axis_name="c", subcore_axis_name="s", num_cores=NC, num_subcores=NS)


@functools.partial(
    pl.kernel,
    out_type=jax.ShapeDtypeStruct((2 * NPAD, HALF), jnp.float32),
    mesh=_sc_mesh,
    compiler_params=pltpu.CompilerParams(needs_layout_passes=False),
    scratch_types=[
        pltpu.VMEM((CH,), jnp.int32),
        pltpu.VMEM((CH,), jnp.int32),
        pltpu.VMEM((CH,), jnp.int32),
        pltpu.VMEM((CH,), jnp.float32),
        pltpu.VMEM((CH,), jnp.int32),
        pltpu.VMEM((CH, HALF), jnp.float32),
        pltpu.VMEM((RB, HALF), jnp.float32),
        pltpu.VMEM_SHARED((NPAD, HALF), jnp.float32),
        pltpu.SemaphoreType.DMA,
    ],
)
def _sc_edge_kernel(xw_hbm, src_hbm, dst_hbm, rel_hbm, norm_hbm,
                    zeros_hbm, out_hbm,
                    srcb, dstb, relb, normb, gidx, rows, rbuf,
                    hacc, sem):
    cid = lax.axis_index("c")
    sid = lax.axis_index("s")

    pltpu.sync_copy(zeros_hbm, hacc.at[pl.ds(sid * ROWS_PT, ROWS_PT)])
    plsc.subcore_barrier()

    cbase = cid * (R * N)
    ebase = sid * EPT

    def chunk_body(i, carry):
        base = ebase + i * CH
        pltpu.sync_copy(src_hbm.at[pl.ds(base, CH)], srcb)
        pltpu.sync_copy(dst_hbm.at[pl.ds(base, CH)], dstb)
        pltpu.sync_copy(rel_hbm.at[pl.ds(base, CH)], relb)
        pltpu.sync_copy(norm_hbm.at[pl.ds(base, CH)], normb)
        for j in range(CH // 16):
            sl = pl.ds(j * 16, 16)
            gidx[sl] = cbase + relb[sl] * N + srcb[sl]
        pltpu.async_copy(xw_hbm.at[gidx], rows, sem).wait()

        def group_body(g, c2):
            cv = normb[pl.ds(g * 16, 16)]
            rbase = g * 16
            for i2 in range(16):
                ce = cv[i2]
                for k in range(HALF // 16):
                    ksl = pl.ds(k * 16, 16)
                    rows[rbase + i2, ksl] = rows[rbase + i2, ksl] * ce
            return c2

        lax.fori_loop(0, CH // 16, group_body, 0)
        pltpu.sync_copy(rows, hacc.at[dstb], add=True)
        return carry

    lax.fori_loop(0, NCHUNK, chunk_body, 0)
    plsc.subcore_barrier()

    for t in range(ROWS_PT // RB):
        r0 = sid * ROWS_PT + t * RB
        pltpu.sync_copy(hacc.at[pl.ds(r0, RB)], rbuf)

        def relu_body(i, c2):
            for k in range(HALF // 16):
                ksl = pl.ds(k * 16, 16)
                rbuf[i, ksl] = jnp.maximum(rbuf[i, ksl], 0.0)
            return c2

        lax.fori_loop(0, RB, relu_body, 0)
        pltpu.sync_copy(rbuf, out_hbm.at[pl.ds(cid * NPAD + r0, RB)])


def kernel(x, edge_index, rel_type, norm, weight, gate_weight):
    xw = _projection(x, weight, gate_weight)
    xw_flat = xw.reshape(2 * R * N, HALF)
    src = edge_index[0]
    dst = edge_index[1]
    normf = norm.reshape(E)
    zeros = jnp.zeros((ROWS_PT, HALF), jnp.float32)
    out = _sc_edge_kernel(xw_flat, src, dst, rel_type, normf, zeros)
    h2 = out.reshape(2, NPAD, HALF)
    return jnp.concatenate([h2[0, :N], h2[1, :N]], axis=1)


# speedup vs baseline: 45.1452x; 2.5189x over previous
import functools

import jax
import jax.numpy as jnp
from jax import lax
from jax.experimental import pallas as pl
from jax.experimental.pallas import tpu as pltpu
from jax.experimental.pallas import tpu_sc as plsc

N = 10000
E = 320000
F = 128
O = 256
R = 8
HALF = O // 2

NC = 2
NS = 16
EPT = E // NS
CH = 80
BLK = 4000
CPB = BLK // CH
NPB = CPB // 2
NBLK = EPT // BLK
NPAD = 10240
ROWS_PT = NPAD // NS

BN = 1000
NB = N // BN


def _proj_body(x_ref, w_ref, gw_ref, xw_ref):
    xb = x_ref[...]
    w = w_ref[0]
    y = jnp.dot(xb, w, preferred_element_type=jnp.float32)
    gw = gw_ref[0]
    gz = jnp.dot(xb, gw, preferred_element_type=jnp.float32)
    y = y * jax.nn.sigmoid(gz)
    xw_ref[0, 0] = y[:, :HALF]
    xw_ref[1, 0] = y[:, HALF:]


def _projection(x, weight, gate_weight):
    return pl.pallas_call(
        _proj_body,
        grid=(NB, R),
        in_specs=[
            pl.BlockSpec((BN, F), lambda nb, r: (nb, 0)),
            pl.BlockSpec((1, F, O), lambda nb, r: (r, 0, 0)),
            pl.BlockSpec((1, F, 1), lambda nb, r: (r, 0, 0)),
        ],
        out_specs=pl.BlockSpec((2, 1, BN, HALF), lambda nb, r: (0, r, nb, 0)),
        out_shape=jax.ShapeDtypeStruct((2, R, N, HALF), jnp.float32),
    )(x, weight, gate_weight)


_sc_mesh = plsc.VectorSubcoreMesh(
    core_axis_name="c", subcore_axis_name="s", num_cores=NC, num_subcores=NS)


@functools.partial(
    pl.kernel,
    out_type=jax.ShapeDtypeStruct((2 * NPAD, HALF), jnp.float32),
    mesh=_sc_mesh,
    compiler_params=pltpu.CompilerParams(needs_layout_passes=False),
    scratch_types=[
        pltpu.VMEM((BLK,), jnp.int32),
        pltpu.VMEM((BLK,), jnp.int32),
        pltpu.VMEM((BLK,), jnp.int32),
        pltpu.VMEM((BLK,), jnp.int32),
        pltpu.VMEM((BLK,), jnp.float32),
        pltpu.VMEM((BLK,), jnp.float32),
        pltpu.VMEM((CH,), jnp.int32),
        pltpu.VMEM((CH,), jnp.int32),
        pltpu.VMEM((CH,), jnp.int32),
        pltpu.VMEM((CH,), jnp.int32),
        pltpu.VMEM((CH, HALF), jnp.float32),
        pltpu.VMEM((CH, HALF), jnp.float32),
        pltpu.VMEM_SHARED((NPAD, HALF), jnp.float32),
        pltpu.SemaphoreType.DMA,
        pltpu.SemaphoreType.DMA,
        pltpu.SemaphoreType.DMA,
    ],
)
def _sc_edge_kernel(xw_hbm, gid_hbm, dst_hbm, norm_hbm,
                    zeros_hbm, out_hbm,
                    gidb0, gidb1, dstb0, dstb1, normb0, normb1,
                    gx0, gx1, dx0, dx1, rows0, rows1,
                    hacc, sem0, sem1, msem):
    cid = lax.axis_index("c")
    sid = lax.axis_index("s")

    pltpu.sync_copy(zeros_hbm, hacc.at[pl.ds(sid * ROWS_PT, ROWS_PT)])

    cbase = cid * (R * N)
    ebase = sid * EPT
    meta = [(gidb0, dstb0, normb0), (gidb1, dstb1, normb1)]

    def meta_copies(b):
        esl = pl.ds(ebase + b * BLK, BLK)
        gb, db, nb = meta[b % 2]
        return [
            pltpu.make_async_copy(gid_hbm.at[esl], gb, msem),
            pltpu.make_async_copy(dst_hbm.at[esl], db, msem),
            pltpu.make_async_copy(norm_hbm.at[esl], nb, msem),
        ]

    def stage(mb, ci, gx, dx):
        gb, db, _ = meta[mb]
        off = ci * CH
        for j in range(CH // 16):
            sl = pl.ds(j * 16, 16)
            esl = pl.ds(off + j * 16, 16)
            gx[sl] = gb[esl] + cbase
            dx[sl] = db[esl]

    def scale_scatter(mb, ci, rows, dx):
        nb = meta[mb][2]
        off = ci * CH

        @plsc.parallel_loop(0, CH // 16, 1, unroll=2)
        def _(g):
            cv = nb[pl.ds(off + g * 16, 16)]
            rbase = g * 16
            for i2 in range(16):
                ce = cv[i2]
                for k in range(HALF // 16):
                    ksl = pl.ds(k * 16, 16)
                    rows[rbase + i2, ksl] = rows[rbase + i2, ksl] * ce

        pltpu.sync_copy(rows, hacc.at[dx], add=True)

    for c in meta_copies(0):
        c.start()
        c.wait()

    for b in range(NBLK):
        mb = b % 2
        if b + 1 < NBLK:
            for c in meta_copies(b + 1):
                c.start()

        stage(mb, 0, gx0, dx0)
        pltpu.async_copy(xw_hbm.at[gx0], rows0, sem0)

        def pair_body(p, carry):
            c0 = p * 2
            stage(mb, c0 + 1, gx1, dx1)
            pltpu.async_copy(xw_hbm.at[gx1], rows1, sem1)
            pltpu.make_async_copy(xw_hbm.at[gx0], rows0, sem0).wait()
            scale_scatter(mb, c0, rows0, dx0)

            @pl.when(p < NPB - 1)
            def _():
                stage(mb, c0 + 2, gx0, dx0)
                pltpu.async_copy(xw_hbm.at[gx0], rows0, sem0)

            pltpu.make_async_copy(xw_hbm.at[gx1], rows1, sem1).wait()
            scale_scatter(mb, c0 + 1, rows1, dx1)
            return carry

        lax.fori_loop(0, NPB, pair_body, 0)

        if b + 1 < NBLK:
            for c in meta_copies(b + 1):
                c.wait()

    plsc.subcore_barrier()

    for t in range(ROWS_PT // CH):
        r0 = sid * ROWS_PT + t * CH
        pltpu.sync_copy(hacc.at[pl.ds(r0, CH)], rows0)

        @plsc.parallel_loop(0, CH, 1, unroll=2)
        def _(i):
            for k in range(HALF // 16):
                ksl = pl.ds(k * 16, 16)
                rows0[i, ksl] = jnp.maximum(rows0[i, ksl], 0.0)

        pltpu.sync_copy(rows0, out_hbm.at[pl.ds(cid * NPAD + r0, CH)])


def kernel(x, edge_index, rel_type, norm, weight, gate_weight):
    xw = _projection(x, weight, gate_weight)
    xw_flat = xw.reshape(2 * R * N, HALF)
    src = edge_index[0]
    dst = edge_index[1]
    gid = rel_type * N + src
    normf = norm.reshape(E)
    zeros = jnp.zeros((ROWS_PT, HALF), jnp.float32)
    out = _sc_edge_kernel(xw_flat, gid, dst, normf, zeros)
    h2 = out.reshape(2, NPAD, HALF)
    return jnp.concatenate([h2[0, :N], h2[1, :N]], axis=1)


# speedup vs baseline: 53.1711x; 1.1778x over previous
import functools

import jax
import jax.numpy as jnp
from jax import lax
from jax.experimental import pallas as pl
from jax.experimental.pallas import tpu as pltpu
from jax.experimental.pallas import tpu_sc as plsc

N = 10000
E = 320000
F = 128
O = 256
R = 8
HALF = O // 2

NC = 2
NS = 16
EPT = E // NS
CH = 80
BLK = 4000
CPB = BLK // CH
NPB = CPB // 2
NBLK = EPT // BLK
NPAD = 10240
ROWS_PT = NPAD // NS

BN = 1000
NB = N // BN


def _proj_body(x_ref, w_ref, gwt_ref, xw_ref):
    xb = x_ref[...]
    gz = jnp.dot(xb, gwt_ref[...], preferred_element_type=jnp.float32)
    s = jax.nn.sigmoid(gz)
    for r in range(R):
        y = jnp.dot(xb, w_ref[r], preferred_element_type=jnp.float32)
        y = y * s[:, r:r + 1]
        xw_ref[0, r] = y[:, :HALF]
        xw_ref[1, r] = y[:, HALF:]


def _projection(x, weight, gwt):
    return pl.pallas_call(
        _proj_body,
        grid=(NB,),
        in_specs=[
            pl.BlockSpec((BN, F), lambda nb: (nb, 0)),
            pl.BlockSpec((R, F, O), lambda nb: (0, 0, 0)),
            pl.BlockSpec((F, R), lambda nb: (0, 0)),
        ],
        out_specs=pl.BlockSpec((2, R, BN, HALF), lambda nb: (0, 0, nb, 0)),
        out_shape=jax.ShapeDtypeStruct((2, R, N, HALF), jnp.float32),
    )(x, weight, gwt)


_sc_mesh = plsc.VectorSubcoreMesh(
    core_axis_name="c", subcore_axis_name="s", num_cores=NC, num_subcores=NS)


@functools.partial(
    pl.kernel,
    out_type=jax.ShapeDtypeStruct((N, O), jnp.float32),
    mesh=_sc_mesh,
    compiler_params=pltpu.CompilerParams(needs_layout_passes=False),
    scratch_types=[
        pltpu.VMEM((BLK,), jnp.int32),
        pltpu.VMEM((BLK,), jnp.int32),
        pltpu.VMEM((BLK,), jnp.int32),
        pltpu.VMEM((BLK,), jnp.int32),
        pltpu.VMEM((BLK,), jnp.float32),
        pltpu.VMEM((BLK,), jnp.float32),
        pltpu.VMEM((CH,), jnp.int32),
        pltpu.VMEM((CH,), jnp.int32),
        pltpu.VMEM((CH,), jnp.int32),
        pltpu.VMEM((CH,), jnp.int32),
        pltpu.VMEM((CH, HALF), jnp.float32),
        pltpu.VMEM((CH, HALF), jnp.float32),
        pltpu.VMEM_SHARED((NPAD, HALF), jnp.float32),
        pltpu.SemaphoreType.DMA,
        pltpu.SemaphoreType.DMA,
        pltpu.SemaphoreType.DMA,
    ],
)
def _sc_edge_kernel(xw_hbm, gid_hbm, dst_hbm, norm_hbm, out_hbm,
                    gidb0, gidb1, dstb0, dstb1, normb0, normb1,
                    gx0, gx1, dx0, dx1, rows0, rows1,
                    hacc, sem0, sem1, msem):
    cid = lax.axis_index("c")
    sid = lax.axis_index("s")

    @plsc.parallel_loop(0, CH, 1, unroll=2)
    def _(i):
        z = jnp.zeros((16,), jnp.float32)
        for k in range(HALF // 16):
            rows0[i, pl.ds(k * 16, 16)] = z

    for q in range(ROWS_PT // CH):
        pltpu.sync_copy(rows0, hacc.at[pl.ds(sid * ROWS_PT + q * CH, CH)])

    cbase = cid * (R * N)
    ebase = sid * EPT
    meta = [(gidb0, dstb0, normb0), (gidb1, dstb1, normb1)]

    def meta_copies(b):
        esl = pl.ds(ebase + b * BLK, BLK)
        gb, db, nb = meta[b % 2]
        return [
            pltpu.make_async_copy(gid_hbm.at[esl], gb, msem),
            pltpu.make_async_copy(dst_hbm.at[esl], db, msem),
            pltpu.make_async_copy(norm_hbm.at[esl], nb, msem),
        ]

    def stage(mb, ci, gx, dx):
        gb, db, _ = meta[mb]
        off = ci * CH
        for j in range(CH // 16):
            sl = pl.ds(j * 16, 16)
            esl = pl.ds(off + j * 16, 16)
            gx[sl] = gb[esl] + cbase
            dx[sl] = db[esl]

    def scale_scatter(mb, ci, rows, dx):
        nb = meta[mb][2]
        off = ci * CH

        @plsc.parallel_loop(0, CH // 16, 1, unroll=2)
        def _(g):
            cv = nb[pl.ds(off + g * 16, 16)]
            rbase = g * 16
            for i2 in range(16):
                ce = cv[i2]
                for k in range(HALF // 16):
                    ksl = pl.ds(k * 16, 16)
                    rows[rbase + i2, ksl] = rows[rbase + i2, ksl] * ce

        pltpu.sync_copy(rows, hacc.at[dx], add=True)

    for c in meta_copies(0):
        c.start()
        c.wait()

    for b in range(NBLK):
        mb = b % 2
        if b + 1 < NBLK:
            for c in meta_copies(b + 1):
                c.start()

        stage(mb, 0, gx0, dx0)
        pltpu.async_copy(xw_hbm.at[gx0], rows0, sem0)

        def pair_body(p, carry):
            c0 = p * 2
            stage(mb, c0 + 1, gx1, dx1)
            pltpu.async_copy(xw_hbm.at[gx1], rows1, sem1)
            pltpu.make_async_copy(xw_hbm.at[gx0], rows0, sem0).wait()
            scale_scatter(mb, c0, rows0, dx0)

            @pl.when(p < NPB - 1)
            def _():
                stage(mb, c0 + 2, gx0, dx0)
                pltpu.async_copy(xw_hbm.at[gx0], rows0, sem0)

            pltpu.make_async_copy(xw_hbm.at[gx1], rows1, sem1).wait()
            scale_scatter(mb, c0 + 1, rows1, dx1)
            return carry

        lax.fori_loop(0, NPB, pair_body, 0)

        if b + 1 < NBLK:
            for c in meta_copies(b + 1):
                c.wait()

    plsc.subcore_barrier()

    for t in range(ROWS_PT // CH):
        r0 = sid * ROWS_PT + t * CH

        @pl.when(r0 < N)
        def _():
            pltpu.sync_copy(hacc.at[pl.ds(r0, CH)], rows0)

            @plsc.parallel_loop(0, CH, 1, unroll=2)
            def _(i):
                for k in range(HALF // 16):
                    ksl = pl.ds(k * 16, 16)
                    rows0[i, ksl] = jnp.maximum(rows0[i, ksl], 0.0)

            pltpu.sync_copy(
                rows0, out_hbm.at[pl.ds(r0, CH), pl.ds(cid * HALF, HALF)])


def kernel(x, edge_index, rel_type, norm, weight, gate_weight):
    gwt = gate_weight.reshape(R, F).T
    xw = _projection(x, weight, gwt)
    xw_flat = xw.reshape(2 * R * N, HALF)
    src = edge_index[0]
    dst = edge_index[1]
    gid = rel_type * N + src
    normf = norm.reshape(E)
    return _sc_edge_kernel(xw_flat, gid, dst, normf)


# speedup vs baseline: 58.2424x; 1.0954x over previous
import functools

import jax
import jax.numpy as jnp
from jax import lax
from jax.experimental import pallas as pl
from jax.experimental.pallas import tpu as pltpu
from jax.experimental.pallas import tpu_sc as plsc

N = 10000
E = 320000
F = 128
O = 256
R = 8
HALF = O // 2

NC = 2
NS = 16
EPT = E // NS
CH = 80
BLK = 4000
CPB = BLK // CH
NTRI = (CPB - 2) // 3
NBLK = EPT // BLK
NPAD = 10240
ROWS_PT = NPAD // NS

BN = 1000
NB = N // BN


def _proj_body(x_ref, w_ref, gwt_ref, xw_ref):
    xb = x_ref[...]
    gz = jnp.dot(xb, gwt_ref[...], preferred_element_type=jnp.float32)
    s = jax.nn.sigmoid(gz)
    for r in range(R):
        y = jnp.dot(xb, w_ref[r], preferred_element_type=jnp.float32)
        y = y * s[:, r:r + 1]
        xw_ref[0, r] = y[:, :HALF]
        xw_ref[1, r] = y[:, HALF:]


def _projection(x, weight, gwt):
    return pl.pallas_call(
        _proj_body,
        grid=(NB,),
        in_specs=[
            pl.BlockSpec((BN, F), lambda nb: (nb, 0)),
            pl.BlockSpec((R, F, O), lambda nb: (0, 0, 0)),
            pl.BlockSpec((F, R), lambda nb: (0, 0)),
        ],
        out_specs=pl.BlockSpec((2, R, BN, HALF), lambda nb: (0, 0, nb, 0)),
        out_shape=jax.ShapeDtypeStruct((2, R, N, HALF), jnp.float32),
    )(x, weight, gwt)


_sc_mesh = plsc.VectorSubcoreMesh(
    core_axis_name="c", subcore_axis_name="s", num_cores=NC, num_subcores=NS)


@functools.partial(
    pl.kernel,
    out_type=jax.ShapeDtypeStruct((N, O), jnp.float32),
    mesh=_sc_mesh,
    compiler_params=pltpu.CompilerParams(needs_layout_passes=False),
    scratch_types=[
        pltpu.VMEM((BLK,), jnp.int32),
        pltpu.VMEM((BLK,), jnp.int32),
        pltpu.VMEM((BLK,), jnp.float32),
        pltpu.VMEM((CH, HALF), jnp.float32),
        pltpu.VMEM((CH, HALF), jnp.float32),
        pltpu.VMEM((CH, HALF), jnp.float32),
        pltpu.VMEM_SHARED((NPAD, HALF), jnp.float32),
        pltpu.SemaphoreType.DMA,
        pltpu.SemaphoreType.DMA,
        pltpu.SemaphoreType.DMA,
    ],
)
def _sc_edge_kernel(xw_hbm, gid_hbm, dst_hbm, norm_hbm, out_hbm,
                    gidb, dstb, normb, rows0, rows1, rows2,
                    hacc, sem0, sem1, sem2):
    cid = lax.axis_index("c")
    sid = lax.axis_index("s")

    @plsc.parallel_loop(0, CH, 1, unroll=2)
    def _(i):
        z = jnp.zeros((16,), jnp.float32)
        for k in range(HALF // 16):
            rows0[i, pl.ds(k * 16, 16)] = z

    for q in range(ROWS_PT // CH):
        pltpu.sync_copy(rows0, hacc.at[pl.ds(sid * ROWS_PT + q * CH, CH)])

    gbase = cid * E + sid * EPT
    ebase = sid * EPT

    def issue(ci, rows, sem):
        pltpu.async_copy(xw_hbm.at[gidb.at[pl.ds(ci * CH, CH)]], rows, sem)

    def drain(ci, rows, sem):
        pltpu.make_async_copy(
            xw_hbm.at[gidb.at[pl.ds(ci * CH, CH)]], rows, sem).wait()

    def scale_scatter(ci, rows):
        off = ci * CH

        @plsc.parallel_loop(0, CH // 16, 1, unroll=2)
        def _(g):
            cv = normb[pl.ds(off + g * 16, 16)]
            rbase = g * 16
            for i2 in range(16):
                ce = cv[i2]
                for k in range(HALF // 16):
                    ksl = pl.ds(k * 16, 16)
                    rows[rbase + i2, ksl] = rows[rbase + i2, ksl] * ce

        pltpu.sync_copy(rows, hacc.at[dstb.at[pl.ds(off, CH)]], add=True)

    def block_body(b, carry):
        pltpu.sync_copy(gid_hbm.at[pl.ds(gbase + b * BLK, BLK)], gidb)
        pltpu.sync_copy(dst_hbm.at[pl.ds(ebase + b * BLK, BLK)], dstb)
        pltpu.sync_copy(norm_hbm.at[pl.ds(ebase + b * BLK, BLK)], normb)

        issue(0, rows0, sem0)
        issue(1, rows1, sem1)

        def triad(t, carry2):
            c = t * 3
            issue(c + 2, rows2, sem2)
            drain(c, rows0, sem0)
            scale_scatter(c, rows0)
            issue(c + 3, rows0, sem0)
            drain(c + 1, rows1, sem1)
            scale_scatter(c + 1, rows1)
            issue(c + 4, rows1, sem1)
            drain(c + 2, rows2, sem2)
            scale_scatter(c + 2, rows2)
            return carry2

        lax.fori_loop(0, NTRI, triad, 0)

        drain(CPB - 2, rows0, sem0)
        scale_scatter(CPB - 2, rows0)
        drain(CPB - 1, rows1, sem1)
        scale_scatter(CPB - 1, rows1)
        return carry

    lax.fori_loop(0, NBLK, block_body, 0)

    plsc.subcore_barrier()

    for t in range(ROWS_PT // CH):
        r0 = sid * ROWS_PT + t * CH

        @pl.when(r0 < N)
        def _():
            pltpu.sync_copy(hacc.at[pl.ds(r0, CH)], rows0)

            @plsc.parallel_loop(0, CH, 1, unroll=2)
            def _(i):
                for k in range(HALF // 16):
                    ksl = pl.ds(k * 16, 16)
                    rows0[i, ksl] = jnp.maximum(rows0[i, ksl], 0.0)

            pltpu.sync_copy(
                rows0, out_hbm.at[pl.ds(r0, CH), pl.ds(cid * HALF, HALF)])


def kernel(x, edge_index, rel_type, norm, weight, gate_weight):
    gwt = gate_weight.reshape(R, F).T
    xw = _projection(x, weight, gwt)
    xw_flat = xw.reshape(2 * R * N, HALF)
    src = edge_index[0]
    dst = edge_index[1]
    gid = rel_type * N + src
    gid2 = jnp.concatenate([gid, gid + R * N])
    normf = norm.reshape(E)
    return _sc_edge_kernel(xw_flat, gid2, dst, normf)
